# Initial kernel scaffold; baseline (speedup 1.0000x reference)
import jax
import jax.numpy as jnp
from jax.experimental import pallas as pl


def kernel(l_xyz_0, l_xyz_1, l_xyz_2, l_xyz_3, l_feat_0, l_feat_1, l_feat_2, l_feat_3, W2a, g2a, b2a, W2b, g2b, b2b, W1a, g1a, b1a, W1b, g1b, b1b, W0a, g0a, b0a, W0b, g0b, b0b):
    raise NotImplementedError("write your pallas kernel here")


# speedup vs baseline: 21.4933x; 21.4933x over previous
import functools

import jax
import jax.numpy as jnp
from jax.experimental import pallas as pl


def _k1_body(ut_ref, kn_ref, kf_ref, uf_ref, wau_ref, wai_ref,
             y1_ref, st_ref):
    b = pl.program_id(0)
    i = pl.program_id(1)

    ut = ut_ref[0]
    kxyz = kn_ref[0]
    dx = kxyz[:, 0:1] - ut[0:1, :]
    dy = kxyz[:, 1:2] - ut[1:2, :]
    dz = kxyz[:, 2:3] - ut[2:3, :]
    d2 = (dx * dx + dy * dy) + dz * dz

    m1 = jnp.min(d2, axis=0, keepdims=True)
    d2a = jnp.where(d2 > m1, d2, jnp.inf)
    m2 = jnp.min(d2a, axis=0, keepdims=True)
    d2b = jnp.where(d2a > m2, d2a, jnp.inf)
    m3 = jnp.min(d2b, axis=0, keepdims=True)

    w = jnp.where(d2 <= m3, 1.0 / (d2 + 1e-8), 0.0)
    w = w / jnp.sum(w, axis=0, keepdims=True)

    interp = jnp.dot(kf_ref[0], w, preferred_element_type=jnp.float32)
    y1 = (jnp.dot(wau_ref[...], uf_ref[0], preferred_element_type=jnp.float32)
          + jnp.dot(wai_ref[...], interp, preferred_element_type=jnp.float32))
    y1_ref[0] = y1

    @pl.when((b == 0) & (i == 0))
    def _():
        st_ref[...] = jnp.zeros_like(st_ref)
    st_ref[:, 0:1] += jnp.sum(y1, axis=1, keepdims=True)
    st_ref[:, 1:2] += jnp.sum(y1 * y1, axis=1, keepdims=True)


def _k2_body(y1_ref, st1_ref, ga_ref, ba_ref, wb_ref, y2_ref, st2_ref,
             *, inv_n):
    b = pl.program_id(0)
    i = pl.program_id(1)

    mean = st1_ref[:, 0:1] * inv_n
    var = st1_ref[:, 1:2] * inv_n - mean * mean
    scale = ga_ref[...] * jax.lax.rsqrt(var + 1e-5)
    x = jnp.maximum((y1_ref[0] - mean) * scale + ba_ref[...], 0.0)
    y2 = jnp.dot(wb_ref[...], x, preferred_element_type=jnp.float32)
    y2_ref[0] = y2

    @pl.when((b == 0) & (i == 0))
    def _():
        st2_ref[...] = jnp.zeros_like(st2_ref)
    st2_ref[:, 0:1] += jnp.sum(y2, axis=1, keepdims=True)
    st2_ref[:, 1:2] += jnp.sum(y2 * y2, axis=1, keepdims=True)


def _k3_body(y2_ref, st2_ref, gb_ref, bb_ref, out_ref, *, inv_n):
    mean = st2_ref[:, 0:1] * inv_n
    var = st2_ref[:, 1:2] * inv_n - mean * mean
    scale = gb_ref[...] * jax.lax.rsqrt(var + 1e-5)
    out_ref[0] = jnp.maximum((y2_ref[0] - mean) * scale + bb_ref[...], 0.0)


def _fp_level(unknown, known, unk_f, kn_f, Wa, ga, ba, Wb, gb, bb, bn):
    B, n, _ = unknown.shape
    m = known.shape[1]
    c_unk = unk_f.shape[1]
    c_kn = kn_f.shape[1]
    c_mid = Wa.shape[0]
    c_out = Wb.shape[0]
    nb = n // bn
    inv_n = 1.0 / (B * n)

    ut = jnp.transpose(unknown, (0, 2, 1))
    wau = Wa[:, :c_unk]
    wai = Wa[:, c_unk:]

    y1, st1 = pl.pallas_call(
        _k1_body,
        grid=(B, nb),
        in_specs=[
            pl.BlockSpec((1, 3, bn), lambda b, i: (b, 0, i)),
            pl.BlockSpec((1, m, 3), lambda b, i: (b, 0, 0)),
            pl.BlockSpec((1, c_kn, m), lambda b, i: (b, 0, 0)),
            pl.BlockSpec((1, c_unk, bn), lambda b, i: (b, 0, i)),
            pl.BlockSpec((c_mid, c_unk), lambda b, i: (0, 0)),
            pl.BlockSpec((c_mid, c_kn), lambda b, i: (0, 0)),
        ],
        out_specs=[
            pl.BlockSpec((1, c_mid, bn), lambda b, i: (b, 0, i)),
            pl.BlockSpec((c_mid, 2), lambda b, i: (0, 0)),
        ],
        out_shape=[
            jax.ShapeDtypeStruct((B, c_mid, n), jnp.float32),
            jax.ShapeDtypeStruct((c_mid, 2), jnp.float32),
        ],
    )(ut, known, kn_f, unk_f, wau, wai)

    y2, st2 = pl.pallas_call(
        functools.partial(_k2_body, inv_n=inv_n),
        grid=(B, nb),
        in_specs=[
            pl.BlockSpec((1, c_mid, bn), lambda b, i: (b, 0, i)),
            pl.BlockSpec((c_mid, 2), lambda b, i: (0, 0)),
            pl.BlockSpec((c_mid, 1), lambda b, i: (0, 0)),
            pl.BlockSpec((c_mid, 1), lambda b, i: (0, 0)),
            pl.BlockSpec((c_out, c_mid), lambda b, i: (0, 0)),
        ],
        out_specs=[
            pl.BlockSpec((1, c_out, bn), lambda b, i: (b, 0, i)),
            pl.BlockSpec((c_out, 2), lambda b, i: (0, 0)),
        ],
        out_shape=[
            jax.ShapeDtypeStruct((B, c_out, n), jnp.float32),
            jax.ShapeDtypeStruct((c_out, 2), jnp.float32),
        ],
    )(y1, st1, ga.reshape(c_mid, 1), ba.reshape(c_mid, 1), Wb)

    out = pl.pallas_call(
        functools.partial(_k3_body, inv_n=inv_n),
        grid=(B, nb),
        in_specs=[
            pl.BlockSpec((1, c_out, bn), lambda b, i: (b, 0, i)),
            pl.BlockSpec((c_out, 2), lambda b, i: (0, 0)),
            pl.BlockSpec((c_out, 1), lambda b, i: (0, 0)),
            pl.BlockSpec((c_out, 1), lambda b, i: (0, 0)),
        ],
        out_specs=pl.BlockSpec((1, c_out, bn), lambda b, i: (b, 0, i)),
        out_shape=jax.ShapeDtypeStruct((B, c_out, n), jnp.float32),
    )(y2, st2, gb.reshape(c_out, 1), bb.reshape(c_out, 1))
    return out


def kernel(l_xyz_0, l_xyz_1, l_xyz_2, l_xyz_3, l_feat_0, l_feat_1,
           l_feat_2, l_feat_3, W2a, g2a, b2a, W2b, g2b, b2b, W1a, g1a,
           b1a, W1b, g1b, b1b, W0a, g0a, b0a, W0b, g0b, b0b):
    f2 = _fp_level(l_xyz_2, l_xyz_3, l_feat_2, l_feat_3,
                   W2a, g2a, b2a, W2b, g2b, b2b, bn=256)
    f1 = _fp_level(l_xyz_1, l_xyz_2, l_feat_1, f2,
                   W1a, g1a, b1a, W1b, g1b, b1b, bn=512)
    f0 = _fp_level(l_xyz_0, l_xyz_1, l_feat_0, f1,
                   W0a, g0a, b0a, W0b, g0b, b0b, bn=512)
    return f0


# speedup vs baseline: 29.7684x; 1.3850x over previous
import functools

import jax
import jax.numpy as jnp
from jax.experimental import pallas as pl


def _k1_body(*refs, inv_np):
    if inv_np is None:
        (ut_ref, kn_ref, kf_ref, uf_ref, wau_ref, wai_ref,
         y1_ref, st_ref) = refs
    else:
        (ut_ref, kn_ref, kf_ref, stp_ref, gp_ref, bp_ref, uf_ref,
         wau_ref, wai_ref, y1_ref, st_ref) = refs
    b = pl.program_id(0)
    i = pl.program_id(1)

    ut = ut_ref[0]
    kxyz = kn_ref[0]
    dx = kxyz[:, 0:1] - ut[0:1, :]
    dy = kxyz[:, 1:2] - ut[1:2, :]
    dz = kxyz[:, 2:3] - ut[2:3, :]
    d2 = (dx * dx + dy * dy) + dz * dz

    m1 = jnp.min(d2, axis=0, keepdims=True)
    d2a = jnp.where(d2 > m1, d2, jnp.inf)
    m2 = jnp.min(d2a, axis=0, keepdims=True)
    d2b = jnp.where(d2a > m2, d2a, jnp.inf)
    m3 = jnp.min(d2b, axis=0, keepdims=True)

    w = jnp.where(d2 <= m3, 1.0 / (d2 + 1e-8), 0.0)
    winv = 1.0 / jnp.sum(w, axis=0, keepdims=True)

    kf = kf_ref[0]
    if inv_np is not None:
        meanp = stp_ref[:, 0:1] * inv_np
        varp = stp_ref[:, 1:2] * inv_np - meanp * meanp
        scalep = gp_ref[...] * jax.lax.rsqrt(varp + 1e-5)
        kf = jnp.maximum((kf - meanp) * scalep + bp_ref[...], 0.0)

    interp = jnp.dot(kf, w, preferred_element_type=jnp.float32) * winv
    y1 = (jnp.dot(wau_ref[...], uf_ref[0], preferred_element_type=jnp.float32)
          + jnp.dot(wai_ref[...], interp, preferred_element_type=jnp.float32))
    y1_ref[0] = y1

    @pl.when((b == 0) & (i == 0))
    def _():
        st_ref[...] = jnp.zeros_like(st_ref)
    st_ref[:, 0:1] += jnp.sum(y1, axis=1, keepdims=True)
    st_ref[:, 1:2] += jnp.sum(y1 * y1, axis=1, keepdims=True)


def _k2_body(y1_ref, st1_ref, ga_ref, ba_ref, wb_ref, y2_ref, st2_ref,
             *, inv_n):
    b = pl.program_id(0)
    i = pl.program_id(1)

    mean = st1_ref[:, 0:1] * inv_n
    var = st1_ref[:, 1:2] * inv_n - mean * mean
    scale = ga_ref[...] * jax.lax.rsqrt(var + 1e-5)
    x = jnp.maximum((y1_ref[0] - mean) * scale + ba_ref[...], 0.0)
    y2 = jnp.dot(wb_ref[...], x, preferred_element_type=jnp.float32)
    y2_ref[0] = y2

    @pl.when((b == 0) & (i == 0))
    def _():
        st2_ref[...] = jnp.zeros_like(st2_ref)
    st2_ref[:, 0:1] += jnp.sum(y2, axis=1, keepdims=True)
    st2_ref[:, 1:2] += jnp.sum(y2 * y2, axis=1, keepdims=True)


def _k3_body(y2_ref, st2_ref, gb_ref, bb_ref, out_ref, *, inv_n):
    mean = st2_ref[:, 0:1] * inv_n
    var = st2_ref[:, 1:2] * inv_n - mean * mean
    scale = gb_ref[...] * jax.lax.rsqrt(var + 1e-5)
    out_ref[0] = jnp.maximum((y2_ref[0] - mean) * scale + bb_ref[...], 0.0)


def _fp_level(unknown, known, unk_f, kn_f_raw, prev_bn,
              Wa, ga, ba, Wb, gb, bb, bn):
    B, n, _ = unknown.shape
    m = known.shape[1]
    c_unk = unk_f.shape[1]
    c_kn = kn_f_raw.shape[1]
    c_mid = Wa.shape[0]
    nb = n // bn
    inv_n = 1.0 / (B * n)
    c_out = Wb.shape[0]

    ut = jnp.transpose(unknown, (0, 2, 1))
    wau = Wa[:, :c_unk]
    wai = Wa[:, c_unk:]

    in_specs = [
        pl.BlockSpec((1, 3, bn), lambda b, i: (b, 0, i)),
        pl.BlockSpec((1, m, 3), lambda b, i: (b, 0, 0)),
        pl.BlockSpec((1, c_kn, m), lambda b, i: (b, 0, 0)),
    ]
    args = [ut, known, kn_f_raw]
    if prev_bn is None:
        inv_np = None
    else:
        stp, gp, bp, inv_np = prev_bn
        in_specs += [
            pl.BlockSpec((c_kn, 2), lambda b, i: (0, 0)),
            pl.BlockSpec((c_kn, 1), lambda b, i: (0, 0)),
            pl.BlockSpec((c_kn, 1), lambda b, i: (0, 0)),
        ]
        args += [stp, gp.reshape(c_kn, 1), bp.reshape(c_kn, 1)]
    in_specs += [
        pl.BlockSpec((1, c_unk, bn), lambda b, i: (b, 0, i)),
        pl.BlockSpec((c_mid, c_unk), lambda b, i: (0, 0)),
        pl.BlockSpec((c_mid, c_kn), lambda b, i: (0, 0)),
    ]
    args += [unk_f, wau, wai]

    y1, st1 = pl.pallas_call(
        functools.partial(_k1_body, inv_np=inv_np),
        grid=(B, nb),
        in_specs=in_specs,
        out_specs=[
            pl.BlockSpec((1, c_mid, bn), lambda b, i: (b, 0, i)),
            pl.BlockSpec((c_mid, 2), lambda b, i: (0, 0)),
        ],
        out_shape=[
            jax.ShapeDtypeStruct((B, c_mid, n), jnp.float32),
            jax.ShapeDtypeStruct((c_mid, 2), jnp.float32),
        ],
    )(*args)

    y2, st2 = pl.pallas_call(
        functools.partial(_k2_body, inv_n=inv_n),
        grid=(B, nb),
        in_specs=[
            pl.BlockSpec((1, c_mid, bn), lambda b, i: (b, 0, i)),
            pl.BlockSpec((c_mid, 2), lambda b, i: (0, 0)),
            pl.BlockSpec((c_mid, 1), lambda b, i: (0, 0)),
            pl.BlockSpec((c_mid, 1), lambda b, i: (0, 0)),
            pl.BlockSpec((c_out, c_mid), lambda b, i: (0, 0)),
        ],
        out_specs=[
            pl.BlockSpec((1, c_out, bn), lambda b, i: (b, 0, i)),
            pl.BlockSpec((c_out, 2), lambda b, i: (0, 0)),
        ],
        out_shape=[
            jax.ShapeDtypeStruct((B, c_out, n), jnp.float32),
            jax.ShapeDtypeStruct((c_out, 2), jnp.float32),
        ],
    )(y1, st1, ga.reshape(c_mid, 1), ba.reshape(c_mid, 1), Wb)
    return y2, st2


def kernel(l_xyz_0, l_xyz_1, l_xyz_2, l_xyz_3, l_feat_0, l_feat_1,
           l_feat_2, l_feat_3, W2a, g2a, b2a, W2b, g2b, b2b, W1a, g1a,
           b1a, W1b, g1b, b1b, W0a, g0a, b0a, W0b, g0b, b0b):
    B = l_xyz_0.shape[0]
    y2_2, st2_2 = _fp_level(l_xyz_2, l_xyz_3, l_feat_2, l_feat_3, None,
                            W2a, g2a, b2a, W2b, g2b, b2b, bn=256)
    y2_1, st2_1 = _fp_level(l_xyz_1, l_xyz_2, l_feat_1, y2_2,
                            (st2_2, g2b, b2b, 1.0 / (B * 256)),
                            W1a, g1a, b1a, W1b, g1b, b1b, bn=1024)
    y2_0, st2_0 = _fp_level(l_xyz_0, l_xyz_1, l_feat_0, y2_1,
                            (st2_1, g1b, b1b, 1.0 / (B * 1024)),
                            W0a, g0a, b0a, W0b, g0b, b0b, bn=1024)

    n0, c0 = 4096, W0b.shape[0]
    out = pl.pallas_call(
        functools.partial(_k3_body, inv_n=1.0 / (B * n0)),
        grid=(B, n0 // 1024),
        in_specs=[
            pl.BlockSpec((1, c0, 1024), lambda b, i: (b, 0, i)),
            pl.BlockSpec((c0, 2), lambda b, i: (0, 0)),
            pl.BlockSpec((c0, 1), lambda b, i: (0, 0)),
            pl.BlockSpec((c0, 1), lambda b, i: (0, 0)),
        ],
        out_specs=pl.BlockSpec((1, c0, 1024), lambda b, i: (b, 0, i)),
        out_shape=jax.ShapeDtypeStruct((B, c0, n0), jnp.float32),
    )(y2_0, st2_0, g0b.reshape(c0, 1), b0b.reshape(c0, 1))
    return out
